# baseline (speedup 1.0000x reference)
import functools

import jax
import jax.numpy as jnp
from jax import lax
from jax.experimental import pallas as pl
from jax.experimental.pallas import tpu as pltpu

_E = 8
_TOP_K = 2
_N_GROUP = 4
_TOPK_GROUP = 2
_D = 1024
_I = 1408
_IS = 2816
_T = 2048
_BI = 128
_BIS = 256
_NJ = 11


def _moe_body(x_ref, comb_ref, wg_ref, wu_ref, wd_ref, sg_ref, su_ref, sd_ref,
              out_ref):
    e = pl.program_id(0)
    j = pl.program_id(1)

    @pl.when((e == 0) & (j == 0))
    def _init():
        out_ref[...] = jnp.zeros_like(out_ref)

    x = x_ref[...]

    def mlp_block(wg, wu, wd, c):
        g = lax.dot_general(x, wg, (((1,), (1,)), ((), ())),
                            preferred_element_type=jnp.float32)
        u = lax.dot_general(x, wu, (((1,), (1,)), ((), ())),
                            preferred_element_type=jnp.float32)
        h = (g * jax.nn.sigmoid(g)) * u
        if c is not None:
            h = h * c
        out_ref[...] += lax.dot_general(h, wd, (((1,), (1,)), ((), ())),
                                        preferred_element_type=jnp.float32)

    @pl.when(e < _E)
    def _routed():
        oh = (lax.broadcasted_iota(jnp.int32, (1, _E), 1) == e)
        c = jnp.sum(jnp.where(oh, comb_ref[...], 0.0), axis=1, keepdims=True)
        mlp_block(wg_ref[0], wu_ref[0], wd_ref[0], c)

    @pl.when(e == _E)
    def _shared():
        mlp_block(sg_ref[...], su_ref[...], sd_ref[...], None)


@functools.partial(jax.jit, static_argnames=("interpret",))
def _moe_matmuls(x, comb, w_gate_up, w_down, shared_w_gate_up, shared_w_down,
                 interpret=False):
    grid = (_E + 1, _NJ)
    return pl.pallas_call(
        _moe_body,
        grid=grid,
        in_specs=[
            pl.BlockSpec((_T, _D), lambda e, j: (0, 0)),
            pl.BlockSpec((_T, _E), lambda e, j: (0, 0)),
            pl.BlockSpec((1, _BI, _D),
                         lambda e, j: (jnp.minimum(e, _E - 1), j, 0)),
            pl.BlockSpec((1, _BI, _D),
                         lambda e, j: (jnp.minimum(e, _E - 1), _NJ + j, 0)),
            pl.BlockSpec((1, _D, _BI),
                         lambda e, j: (jnp.minimum(e, _E - 1), 0, j)),
            pl.BlockSpec((_BIS, _D),
                         lambda e, j: (jnp.where(e == _E, j, 0), 0)),
            pl.BlockSpec((_BIS, _D),
                         lambda e, j: (jnp.where(e == _E, _NJ + j, _NJ), 0)),
            pl.BlockSpec((_D, _BIS),
                         lambda e, j: (0, jnp.where(e == _E, j, 0))),
        ],
        out_specs=pl.BlockSpec((_T, _D), lambda e, j: (0, 0)),
        out_shape=jax.ShapeDtypeStruct((_T, _D), jnp.float32),
        compiler_params=pltpu.CompilerParams(
            dimension_semantics=("arbitrary", "arbitrary"),
            vmem_limit_bytes=120 * 1024 * 1024,
        ),
        interpret=interpret,
    )(x, comb, w_gate_up, w_gate_up, w_down,
      shared_w_gate_up, shared_w_gate_up, shared_w_down)


def _routing_combine(x, gate_w):
    num_tokens = x.shape[0]
    logits = x @ gate_w.T
    scores = jax.nn.softmax(logits, axis=-1)
    group_scores = scores.reshape(num_tokens, _N_GROUP, _E // _N_GROUP).max(axis=-1)
    _, group_idx = jax.lax.top_k(group_scores, _TOPK_GROUP)
    group_mask = jnp.sum(jax.nn.one_hot(group_idx, _N_GROUP, dtype=scores.dtype), axis=1)
    score_mask = jnp.repeat(group_mask, _E // _N_GROUP, axis=1)
    masked_scores = jnp.where(score_mask > 0, scores, 0.0)
    topk_w, topk_ids = jax.lax.top_k(masked_scores, _TOP_K)
    topk_w = topk_w / (jnp.sum(topk_w, axis=-1, keepdims=True) + 1e-20)
    combine = jnp.zeros((num_tokens, _E), dtype=x.dtype).at[
        jnp.arange(num_tokens)[:, None], topk_ids
    ].add(topk_w)
    return combine


def kernel(hidden_states, gate_w, w_gate_up, w_down, shared_w_gate_up, shared_w_down):
    x = hidden_states.reshape(-1, _D)
    comb = _routing_combine(x, gate_w)
    out = _moe_matmuls(x, comb, w_gate_up, w_down, shared_w_gate_up, shared_w_down)
    return out.reshape(hidden_states.shape)


# speedup vs baseline: 1.0541x; 1.0541x over previous
import functools

import jax
import jax.numpy as jnp
from jax import lax
from jax.experimental import pallas as pl
from jax.experimental.pallas import tpu as pltpu

_E = 8
_TOP_K = 2
_N_GROUP = 4
_TOPK_GROUP = 2
_D = 1024
_I = 1408
_IS = 2816
_T = 2048
_BI = 128
_BIS = 256
_NJ = 11


def _moe_body(x_ref, comb_ref, wg_ref, wu_ref, wd_ref, sg_ref, su_ref, sd_ref,
              out_ref):
    e = pl.program_id(0)
    j = pl.program_id(1)

    @pl.when((e == 0) & (j == 0))
    def _init():
        out_ref[...] = jnp.zeros_like(out_ref)

    x = x_ref[...]

    def mlp_block(wg, wu, wd, c):
        g = lax.dot_general(x, wg.astype(jnp.bfloat16), (((1,), (1,)), ((), ())),
                            preferred_element_type=jnp.float32)
        u = lax.dot_general(x, wu.astype(jnp.bfloat16), (((1,), (1,)), ((), ())),
                            preferred_element_type=jnp.float32)
        h = (g * jax.nn.sigmoid(g)) * u
        if c is not None:
            h = h * c
        out_ref[...] += lax.dot_general(h.astype(jnp.bfloat16),
                                        wd.astype(jnp.bfloat16),
                                        (((1,), (1,)), ((), ())),
                                        preferred_element_type=jnp.float32)

    @pl.when(e < _E)
    def _routed():
        oh = (lax.broadcasted_iota(jnp.int32, (1, _E), 1) == e)
        c = jnp.sum(jnp.where(oh, comb_ref[...], 0.0), axis=1, keepdims=True)
        mlp_block(wg_ref[0], wu_ref[0], wd_ref[0], c)

    @pl.when(e == _E)
    def _shared():
        mlp_block(sg_ref[...], su_ref[...], sd_ref[...], None)


@functools.partial(jax.jit, static_argnames=("interpret",))
def _moe_matmuls(x, comb, w_gate_up, w_down, shared_w_gate_up, shared_w_down,
                 interpret=False):
    grid = (_E + 1, _NJ)
    return pl.pallas_call(
        _moe_body,
        grid=grid,
        in_specs=[
            pl.BlockSpec((_T, _D), lambda e, j: (0, 0)),
            pl.BlockSpec((_T, _E), lambda e, j: (0, 0)),
            pl.BlockSpec((1, _BI, _D),
                         lambda e, j: (jnp.minimum(e, _E - 1), j, 0)),
            pl.BlockSpec((1, _BI, _D),
                         lambda e, j: (jnp.minimum(e, _E - 1), _NJ + j, 0)),
            pl.BlockSpec((1, _D, _BI),
                         lambda e, j: (jnp.minimum(e, _E - 1), 0, j)),
            pl.BlockSpec((_BIS, _D),
                         lambda e, j: (jnp.where(e == _E, j, 0), 0)),
            pl.BlockSpec((_BIS, _D),
                         lambda e, j: (jnp.where(e == _E, _NJ + j, _NJ), 0)),
            pl.BlockSpec((_D, _BIS),
                         lambda e, j: (0, jnp.where(e == _E, j, 0))),
        ],
        out_specs=pl.BlockSpec((_T, _D), lambda e, j: (0, 0)),
        out_shape=jax.ShapeDtypeStruct((_T, _D), jnp.float32),
        compiler_params=pltpu.CompilerParams(
            dimension_semantics=("arbitrary", "arbitrary"),
            vmem_limit_bytes=120 * 1024 * 1024,
        ),
        interpret=interpret,
    )(x.astype(jnp.bfloat16), comb, w_gate_up, w_gate_up, w_down,
      shared_w_gate_up, shared_w_gate_up, shared_w_down)


def _routing_combine(x, gate_w):
    num_tokens = x.shape[0]
    logits = x @ gate_w.T
    scores = jax.nn.softmax(logits, axis=-1)
    group_scores = scores.reshape(num_tokens, _N_GROUP, _E // _N_GROUP).max(axis=-1)
    _, group_idx = jax.lax.top_k(group_scores, _TOPK_GROUP)
    group_mask = jnp.sum(jax.nn.one_hot(group_idx, _N_GROUP, dtype=scores.dtype), axis=1)
    score_mask = jnp.repeat(group_mask, _E // _N_GROUP, axis=1)
    masked_scores = jnp.where(score_mask > 0, scores, 0.0)
    topk_w, topk_ids = jax.lax.top_k(masked_scores, _TOP_K)
    topk_w = topk_w / (jnp.sum(topk_w, axis=-1, keepdims=True) + 1e-20)
    combine = jnp.zeros((num_tokens, _E), dtype=x.dtype).at[
        jnp.arange(num_tokens)[:, None], topk_ids
    ].add(topk_w)
    return combine


def kernel(hidden_states, gate_w, w_gate_up, w_down, shared_w_gate_up, shared_w_down):
    x = hidden_states.reshape(-1, _D)
    comb = _routing_combine(x, gate_w)
    out = _moe_matmuls(x, comb, w_gate_up, w_down, shared_w_gate_up, shared_w_down)
    return out.reshape(hidden_states.shape)
